# Initial kernel scaffold; baseline (speedup 1.0000x reference)
import jax
import jax.numpy as jnp
from jax.experimental import pallas as pl


def kernel(x_author, x_paper, params, edge_writes, edge_rev):
    raise NotImplementedError("write your pallas kernel here")


# speedup vs baseline: 51.6051x; 51.6051x over previous
import functools

import jax
import jax.numpy as jnp
from jax import lax
from jax.experimental import pallas as pl
from jax.experimental.pallas import tpu as pltpu, tpu_sc as plsc

N = 10000
NPAD = 10240
E = 320000
H = 8
D = 16
F = 128
FH = 64
CH = 128
NCH = 157
EPT = NCH * CH
NT = 16
ROWS_PT = NPAD // NT


def _expand_mat():
    col_h = lax.broadcasted_iota(jnp.int32, (16, F), 1) // D
    row = lax.broadcasted_iota(jnp.int32, (16, F), 0)
    return (col_h == row).astype(jnp.float32)


def _norm_relu(u, den):
    denx = jnp.dot(den, _expand_mat(), preferred_element_type=jnp.float32)
    return jnp.maximum(u / jnp.maximum(denx, 1e-16), 0.0)


def _proj_body(do_norm, u_ref, den_ref, w_ref, b_ref, ss_ref, sd_ref,
               h_ref, as_ref, ad_ref):
    u = jnp.concatenate([u_ref[0, 0], u_ref[0, 1]], axis=1)
    if do_norm:
        u = _norm_relu(u, den_ref[0])
    h = jnp.dot(u, w_ref[0], preferred_element_type=jnp.float32) + b_ref[0]
    h_ref[0, 0] = h[:, :FH]
    h_ref[0, 1] = h[:, FH:]
    as_ref[0] = jnp.dot(h, ss_ref[0], preferred_element_type=jnp.float32)
    ad_ref[0] = jnp.dot(h, sd_ref[0], preferred_element_type=jnp.float32)


def _proj(u4, den2, w2, b2, ss2, sd2, do_norm):
    R = 256
    if do_norm:
        umap = lambda t, r: (1 - t, 0, r, 0)
        dmap = lambda t, r: (1 - t, r, 0)
    else:
        umap = lambda t, r: (t, 0, r, 0)
        dmap = lambda t, r: (t, r, 0)
    return pl.pallas_call(
        functools.partial(_proj_body, do_norm),
        grid=(2, NPAD // R),
        in_specs=[
            pl.BlockSpec((1, 2, R, FH), umap),
            pl.BlockSpec((1, R, 16), dmap),
            pl.BlockSpec((1, F, F), lambda t, r: (t, 0, 0)),
            pl.BlockSpec((1, 1, F), lambda t, r: (t, 0, 0)),
            pl.BlockSpec((1, F, 16), lambda t, r: (t, 0, 0)),
            pl.BlockSpec((1, F, 16), lambda t, r: (t, 0, 0)),
        ],
        out_specs=[
            pl.BlockSpec((1, 2, R, FH), lambda t, r: (t, 0, r, 0)),
            pl.BlockSpec((1, R, 16), lambda t, r: (t, r, 0)),
            pl.BlockSpec((1, R, 16), lambda t, r: (t, r, 0)),
        ],
        out_shape=[
            jax.ShapeDtypeStruct((2, 2, NPAD, FH), jnp.float32),
            jax.ShapeDtypeStruct((2, NPAD, 16), jnp.float32),
            jax.ShapeDtypeStruct((2, NPAD, 16), jnp.float32),
        ],
    )(u4, den2, w2, b2, ss2, sd2)


def _finalize_body(u_ref, den_ref, x_ref):
    u = jnp.concatenate([u_ref[0, 0], u_ref[0, 1]], axis=1)
    x_ref[0] = _norm_relu(u, den_ref[0])


def _finalize(u4, den2):
    R = 256
    return pl.pallas_call(
        _finalize_body,
        grid=(2, NPAD // R),
        in_specs=[
            pl.BlockSpec((1, 2, R, FH), lambda t, r: (1 - t, 0, r, 0)),
            pl.BlockSpec((1, R, 16), lambda t, r: (1 - t, r, 0)),
        ],
        out_specs=pl.BlockSpec((1, R, F), lambda t, r: (t, r, 0)),
        out_shape=jax.ShapeDtypeStruct((2, NPAD, F), jnp.float32),
    )(u4, den2)


def _prop_pair(h4, as2, ad2, src0, dst0, src1, dst1):
    mesh = plsc.VectorSubcoreMesh(core_axis_name="c", subcore_axis_name="s")

    @functools.partial(
        pl.kernel,
        out_type=[
            jax.ShapeDtypeStruct((2, 2, NPAD, FH), jnp.float32),
            jax.ShapeDtypeStruct((2, NPAD, 16), jnp.float32),
        ],
        mesh=mesh,
        compiler_params=pltpu.CompilerParams(use_tc_tiling_on_sc=False),
        scratch_types=[
            pltpu.VMEM((NCH, CH), jnp.int32),
            pltpu.VMEM((NCH, CH), jnp.int32),
            pltpu.VMEM((CH, 16), jnp.float32),
            pltpu.VMEM((CH, 16), jnp.float32),
            pltpu.VMEM((CH, FH), jnp.float32),
            pltpu.VMEM((CH, 16), jnp.float32),
            pltpu.VMEM((CH, FH), jnp.float32),
            pltpu.VMEM((CH, 16), jnp.float32),
            pltpu.VMEM_SHARED((NPAD, FH), jnp.float32),
            pltpu.VMEM_SHARED((NPAD, 16), jnp.float32),
            pltpu.SemaphoreType.DMA,
        ],
    )
    def k(h_hbm, as_hbm, ad_hbm, s0_hbm, d0_hbm, s1_hbm, d1_hbm,
          u_hbm, den_hbm,
          sidx, didx, asg, adg, hg, ev, zbuf, zbufd, acc_u, acc_d, sem):
        c = lax.axis_index("c")
        s = lax.axis_index("s")
        base = s * ROWS_PT

        def _zero_row(i, _):
            z = jnp.zeros((16,), jnp.float32)
            for k8 in range(FH // 16):
                zbuf[i, pl.ds(k8 * 16, 16)] = z
            zbufd[i, :] = z
            return 0
        lax.fori_loop(0, CH, _zero_row, 0)

        def zero_acc(phase):
            for k5 in range(ROWS_PT // CH):
                pltpu.sync_copy(zbuf, acc_u.at[pl.ds(base + k5 * CH, CH), :])
                if phase == 0:
                    pltpu.sync_copy(zbufd,
                                    acc_d.at[pl.ds(base + k5 * CH, CH), :])

        def run(hsrc, asa, ada, src_e, dst_e, u_out, d_out):
            pltpu.sync_copy(src_e.at[s], sidx)
            pltpu.sync_copy(dst_e.at[s], didx)

            for phase in range(2):
                zero_acc(phase)
                plsc.subcore_barrier()

                def chunk(j, _):
                    si = sidx.at[j]
                    di = didx.at[j]
                    cp1 = pltpu.async_copy(asa.at[si], asg, sem)
                    cp2 = pltpu.async_copy(ada.at[di], adg, sem)
                    cp3 = pltpu.async_copy(hsrc.at[phase].at[si], hg, sem)
                    cp1.wait()
                    cp2.wait()
                    cp3.wait()

                    def edge(cc, _):
                        a = asg[cc, :] + adg[cc, :]
                        e = jnp.exp(jnp.maximum(a, 0.2 * a))
                        if phase == 0:
                            ev[cc, :] = e
                        for hh in range(FH // D):
                            sl = pl.ds(hh * D, D)
                            hg[cc, sl] = hg[cc, sl] * e[phase * (FH // D) + hh]
                        return 0
                    lax.fori_loop(0, CH, edge, 0)

                    if phase == 0:
                        pltpu.sync_copy(ev, acc_d.at[di], add=True)
                    pltpu.sync_copy(hg, acc_u.at[di], add=True)
                    return 0
                lax.fori_loop(0, NCH, chunk, 0)
                plsc.subcore_barrier()

                pltpu.sync_copy(acc_u.at[pl.ds(base, ROWS_PT), :],
                                u_out.at[phase].at[pl.ds(base, ROWS_PT), :])
                if phase == 0:
                    pltpu.sync_copy(acc_d.at[pl.ds(base, ROWS_PT), :],
                                    d_out.at[pl.ds(base, ROWS_PT), :])
                plsc.subcore_barrier()

        @pl.when(c == 0)
        def _():
            run(h_hbm.at[0], as_hbm.at[0], ad_hbm.at[1], s0_hbm, d0_hbm,
                u_hbm.at[0], den_hbm.at[0])

        @pl.when(c == 1)
        def _():
            run(h_hbm.at[1], as_hbm.at[1], ad_hbm.at[0], s1_hbm, d1_hbm,
                u_hbm.at[1], den_hbm.at[1])

    return k(h4, as2, ad2, src0, dst0, src1, dst1)


def _mk_s(a):
    s = jnp.zeros((F, 16), jnp.float32)
    rows = jnp.arange(F)
    return s.at[rows, rows // D].set(a.reshape(F))


def _prep_edges(ei):
    src = jnp.concatenate([ei[0], jnp.zeros((NT * EPT - E,), jnp.int32)])
    dst = jnp.concatenate([ei[1], jnp.full((NT * EPT - E,), N, jnp.int32)])
    return src.reshape(NT, NCH, CH), dst.reshape(NT, NCH, CH)


def _layer_weights(L):
    w2 = jnp.stack([L['Wp_a'], L['Wp_p']])
    b2 = jnp.stack([L['bp_a'], L['bp_p']])[:, None, :]
    ss2 = jnp.stack([_mk_s(L['as_e1']), _mk_s(L['as_e2'])])
    sd2 = jnp.stack([_mk_s(L['ad_e2']), _mk_s(L['ad_e1'])])
    return w2, b2, ss2, sd2


def kernel(x_author, x_paper, params, edge_writes, edge_rev):
    s0, d0 = _prep_edges(edge_writes)
    s1, d1 = _prep_edges(edge_rev)

    pad = ((0, NPAD - N), (0, 0))
    x2 = jnp.stack([jnp.pad(x_author, pad), jnp.pad(x_paper, pad)])
    u4 = x2.reshape(2, NPAD, 2, FH).transpose(0, 2, 1, 3)
    den2 = jnp.zeros((2, NPAD, 16), jnp.float32)

    for li in (1, 2):
        wts = _layer_weights(params['L%d' % li])
        h4, as2, ad2 = _proj(u4, den2, *wts, do_norm=(li == 2))
        u4, den2 = _prop_pair(h4, as2, ad2, s0, d0, s1, d1)

    x2 = _finalize(u4, den2)
    return x2[0, :N], x2[1, :N]
